# baseline (device time: 32495 ns/iter reference)
import jax
import jax.numpy as jnp
from jax import lax
from jax.experimental import pallas as pl
from jax.experimental.pallas import tpu as pltpu

N_DEV = 8
SEQ = 1024
HQ = 8
DH = 128
WIN = 128
D_MODEL = HQ * DH
QBLK = 256
KBLK = QBLK + 2 * WIN
SCALE = 0.08838834764831843


def kernel(x, Wq, K_ext, V_ext, Wo):
    kb = K_ext.reshape(SEQ, D_MODEL).astype(jnp.bfloat16)
    vb = V_ext.reshape(SEQ, D_MODEL).astype(jnp.bfloat16)

    def body(x_ref, wq_ref, k_hbm, v_hbm, wo_ref, out_hbm,
             sendl, sendr, halol, halor, k_full, v_full,
             ctx_scr, out_scr, local_sems, send_sems, recv_sems):
        my = lax.axis_index("i")
        left = lax.rem(my + N_DEV - 1, N_DEV)
        right = lax.rem(my + 1, N_DEV)

        cps = [
            pltpu.make_async_copy(
                k_hbm, k_full.at[WIN:WIN + SEQ, :], local_sems.at[0]),
            pltpu.make_async_copy(
                v_hbm, v_full.at[WIN:WIN + SEQ, :], local_sems.at[0]),
            pltpu.make_async_copy(
                k_hbm.at[SEQ - WIN:, :], sendr.at[0], local_sems.at[1]),
            pltpu.make_async_copy(
                v_hbm.at[SEQ - WIN:, :], sendr.at[1], local_sems.at[1]),
            pltpu.make_async_copy(
                k_hbm.at[:WIN, :], sendl.at[0], local_sems.at[1]),
            pltpu.make_async_copy(
                v_hbm.at[:WIN, :], sendl.at[1], local_sems.at[1]),
        ]
        for cp in cps:
            cp.start()

        barrier_sem = pltpu.get_barrier_semaphore()
        for nbr in (left, right):
            pl.semaphore_signal(
                barrier_sem, inc=1,
                device_id=(nbr,), device_id_type=pl.DeviceIdType.MESH,
            )
        pl.semaphore_wait(barrier_sem, 2)

        for cp in cps[2:]:
            cp.wait()

        rdma_r = pltpu.make_async_remote_copy(
            src_ref=sendr, dst_ref=halol,
            send_sem=send_sems.at[1], recv_sem=recv_sems.at[0],
            device_id=(right,), device_id_type=pl.DeviceIdType.MESH,
        )
        rdma_l = pltpu.make_async_remote_copy(
            src_ref=sendl, dst_ref=halor,
            send_sem=send_sems.at[0], recv_sem=recv_sems.at[1],
            device_id=(left,), device_id_type=pl.DeviceIdType.MESH,
        )
        rdma_r.start()
        rdma_l.start()

        q_bf = (jnp.dot(x_ref[0].astype(jnp.bfloat16),
                        wq_ref[:].astype(jnp.bfloat16),
                        preferred_element_type=jnp.float32)
                * SCALE).astype(jnp.bfloat16)

        cps[0].wait()
        cps[1].wait()
        rdma_r.wait_recv()
        rdma_l.wait_recv()
        k_full[:WIN, :] = halol[0]
        v_full[:WIN, :] = halol[1]
        k_full[WIN + SEQ:, :] = halor[0]
        v_full[WIN + SEQ:, :] = halor[1]
        rdma_r.wait_send()
        rdma_l.wait_send()

        qi = lax.broadcasted_iota(jnp.int32, (QBLK, KBLK), 0)
        ki = lax.broadcasted_iota(jnp.int32, (QBLK, KBLK), 1)
        delta = qi - ki + WIN
        window = (delta >= -WIN) & (delta <= WIN)
        neg = jnp.float32(-1e9)

        for b in range(SEQ // QBLK):
            ki_glob = my * SEQ + b * QBLK - WIN + ki
            maskb = window & (ki_glob >= 0) & (ki_glob < N_DEV * SEQ)
            for h in range(HQ):
                qh = q_bf[b * QBLK:(b + 1) * QBLK, h * DH:(h + 1) * DH]
                kh = k_full[b * QBLK:b * QBLK + KBLK, h * DH:(h + 1) * DH]
                scores = lax.dot_general(
                    qh, kh, (((1,), (1,)), ((), ())),
                    preferred_element_type=jnp.float32,
                )
                p = jnp.exp(jnp.where(maskb, scores, neg))
                s = jnp.sum(p, axis=1, keepdims=True)
                ctx = jnp.dot(
                    p.astype(jnp.bfloat16),
                    v_full[b * QBLK:b * QBLK + KBLK, h * DH:(h + 1) * DH],
                    preferred_element_type=jnp.float32,
                )
                ctx_scr[b * QBLK:(b + 1) * QBLK, h * DH:(h + 1) * DH] = (
                    (ctx / s).astype(jnp.bfloat16))

        out_scr[:] = jnp.dot(ctx_scr[:], wo_ref[:].astype(jnp.bfloat16),
                             preferred_element_type=jnp.float32
                             ).astype(jnp.bfloat16)
        cp_out = pltpu.make_async_copy(out_scr, out_hbm.at[0], local_sems.at[2])
        cp_out.start()
        cp_out.wait()

    out = pl.pallas_call(
        body,
        out_shape=jax.ShapeDtypeStruct((1, SEQ, D_MODEL), jnp.bfloat16),
        in_specs=[
            pl.BlockSpec(memory_space=pltpu.VMEM),
            pl.BlockSpec(memory_space=pltpu.VMEM),
            pl.BlockSpec(memory_space=pltpu.MemorySpace.HBM),
            pl.BlockSpec(memory_space=pltpu.MemorySpace.HBM),
            pl.BlockSpec(memory_space=pltpu.VMEM),
        ],
        out_specs=pl.BlockSpec(memory_space=pltpu.MemorySpace.HBM),
        scratch_shapes=[
            pltpu.VMEM((2, WIN, D_MODEL), jnp.bfloat16),
            pltpu.VMEM((2, WIN, D_MODEL), jnp.bfloat16),
            pltpu.VMEM((2, WIN, D_MODEL), jnp.bfloat16),
            pltpu.VMEM((2, WIN, D_MODEL), jnp.bfloat16),
            pltpu.VMEM((SEQ + 2 * WIN, D_MODEL), jnp.bfloat16),
            pltpu.VMEM((SEQ + 2 * WIN, D_MODEL), jnp.bfloat16),
            pltpu.VMEM((SEQ, D_MODEL), jnp.bfloat16),
            pltpu.VMEM((SEQ, D_MODEL), jnp.bfloat16),
            pltpu.SemaphoreType.DMA((3,)),
            pltpu.SemaphoreType.DMA((2,)),
            pltpu.SemaphoreType.DMA((2,)),
        ],
        compiler_params=pltpu.CompilerParams(collective_id=0),
    )(x, Wq, kb, vb, Wo)
    return out
